# Initial kernel scaffold; baseline (speedup 1.0000x reference)
import jax
import jax.numpy as jnp
from jax.experimental import pallas as pl


def kernel(x, W_emb, b_emb, tok_t, tok_f_real, tok_f_imag, Wt1, bt1, Wt2, bt2, Wf1, bf1, Wf2, bf2):
    raise NotImplementedError("write your pallas kernel here")


# speedup vs baseline: 9.6520x; 9.6520x over previous
import math

import jax
import jax.numpy as jnp
from jax.experimental import pallas as pl

_WINDOW = 24
_T_RATIO = 0.1
_F_RATIO = 0.1


def _pe_table(T, D):
    pos = jnp.arange(T, dtype=jnp.float32)[:, None]
    div = jnp.exp(jnp.arange(0, D, 2, dtype=jnp.float32) * (-(math.log(10000.0) / D)))
    pe = jnp.zeros((T, D), jnp.float32)
    pe = pe.at[:, 0::2].set(jnp.sin(pos * div))
    pe = pe.at[:, 1::2].set(jnp.cos(pos * div))
    return pe


def _win_sum_ref(e, W):
    B, D, T = e.shape
    pad = jnp.pad(e, ((0, 0), (0, 0), (W - 1, W - 1)))
    cs = jnp.cumsum(pad, axis=-1)
    cs = jnp.concatenate([jnp.zeros((B, D, 1), e.dtype), cs], axis=-1)
    out = cs[..., W:] - cs[..., :-W]
    denom = jnp.concatenate(
        [jnp.arange(1, W, dtype=jnp.float32), jnp.full((T,), float(W), jnp.float32)])
    return out / denom


def _gelu(x):
    return 0.5 * x * (1.0 + jax.lax.erf(x * (1.0 / math.sqrt(2.0))))


def _topk_call(vals, k, kpad):
    Bv, N = vals.shape

    def body(v_ref, idx_ref, sel_ref):
        v = v_ref[...]
        iota = jax.lax.broadcasted_iota(jnp.int32, (Bv, N), 1)
        iK = jax.lax.broadcasted_iota(jnp.int32, (Bv, kpad), 1)

        def step(i, carry):
            sel, idxv = carry
            masked = jnp.where(sel > 0, -jnp.inf, v)
            m = jnp.max(masked, axis=1, keepdims=True)
            cand = (masked == m) & (sel <= 0)
            pos = jnp.where(cand, iota, N)
            idx = jnp.min(pos, axis=1, keepdims=True)
            sel = jnp.where(iota == idx, 1.0, sel)
            idxv = jnp.where(iK == i, idx, idxv)
            return sel, idxv

        sel0 = jnp.zeros((Bv, N), jnp.float32)
        idxv0 = jnp.zeros((Bv, kpad), jnp.int32)
        sel, idxv = jax.lax.fori_loop(0, k, step, (sel0, idxv0))
        idx_ref[...] = idxv
        sel_ref[...] = sel

    idx, sel = pl.pallas_call(
        body,
        out_shape=[
            jax.ShapeDtypeStruct((Bv, kpad), jnp.int32),
            jax.ShapeDtypeStruct((Bv, N), jnp.float32),
        ],
    )(vals)
    return idx, sel


def _temporal_call(x2, pe2, mask2, W_emb, b_emb2, tokt2, Wt1, bt12, Wt2, bt22):
    R, C = x2.shape
    D = W_emb.shape[0]
    BLK = 128
    dn = (((1,), (1,)), ((), ()))

    def body(x_ref, pe_ref, m_ref, we_ref, be_ref, tk_ref, w1_ref, b1_ref,
             w2_ref, b2_ref, o_ref):
        e = jax.lax.dot_general(x_ref[...], we_ref[...], dn,
                                preferred_element_type=jnp.float32)
        e = e + be_ref[...] + pe_ref[...]
        mb = m_ref[...] > 0
        mx = jnp.where(mb, tk_ref[...], e)
        h = _gelu(jax.lax.dot_general(mx, w1_ref[...], dn,
                                      preferred_element_type=jnp.float32)
                  + b1_ref[...])
        p = jax.nn.sigmoid(jax.lax.dot_general(h, w2_ref[...], dn,
                                               preferred_element_type=jnp.float32)
                           + b2_ref[...])
        o_ref[...] = jnp.where(mb, mx, p)

    full = lambda shape: pl.BlockSpec(shape, lambda i: (0, 0))
    return pl.pallas_call(
        body,
        grid=(R // BLK,),
        in_specs=[
            pl.BlockSpec((BLK, C), lambda i: (i, 0)),
            pl.BlockSpec((BLK, D), lambda i: (i, 0)),
            pl.BlockSpec((BLK, 1), lambda i: (i, 0)),
            full((D, C)),
            full((1, D)),
            full((1, D)),
            full((D, D)),
            full((1, D)),
            full((D, D)),
            full((1, D)),
        ],
        out_specs=pl.BlockSpec((BLK, D), lambda i: (i, 0)),
        out_shape=jax.ShapeDtypeStruct((R, D), jnp.float32),
    )(x2, pe2, mask2, W_emb, b_emb2, tokt2, Wt1, bt12, Wt2, bt22)


def _freq_call(cr, ci, maskf, tokr, toki, A, Bm, W_emb):
    B, D, Fn = cr.shape
    T = A.shape[1]
    C = W_emb.shape[1]
    dn0 = (((0,), (0,)), ((), ()))

    def body(cr_ref, ci_ref, mf_ref, tr_ref, ti_ref, a_ref, b_ref, we_ref, o_ref):
        mf = mf_ref[0] > 0
        mre = jnp.where(mf, tr_ref[...], cr_ref[0])
        mim = jnp.where(mf, ti_ref[...], ci_ref[0])
        mxdt = (jnp.dot(mre, a_ref[...], preferred_element_type=jnp.float32)
                + jnp.dot(mim, b_ref[...], preferred_element_type=jnp.float32))
        o_ref[0] = jax.lax.dot_general(mxdt, we_ref[...], dn0,
                                       preferred_element_type=jnp.float32)

    full = lambda shape: pl.BlockSpec(shape, lambda b: tuple(0 for _ in shape))
    return pl.pallas_call(
        body,
        grid=(B,),
        in_specs=[
            pl.BlockSpec((1, D, Fn), lambda b: (b, 0, 0)),
            pl.BlockSpec((1, D, Fn), lambda b: (b, 0, 0)),
            pl.BlockSpec((1, 1, Fn), lambda b: (b, 0, 0)),
            full((D, 1)),
            full((D, 1)),
            full((Fn, T)),
            full((Fn, T)),
            full((D, C)),
        ],
        out_specs=pl.BlockSpec((1, T, C), lambda b: (b, 0, 0)),
        out_shape=jax.ShapeDtypeStruct((B, T, C), jnp.float32),
    )(cr, ci, maskf, tokr, toki, A, Bm, W_emb)


def _projf_call(s_col, tm_col, w1row, b1row, w2col, b2s):
    N = s_col.shape[0]
    D = w1row.shape[1]
    R = 256

    def body(s_ref, t_ref, w1_ref, b1_ref, w2_ref, b2_ref, o_ref):
        s = s_ref[...]
        h = _gelu(s * w1_ref[...] + b1_ref[...])
        a = jnp.dot(h, w2_ref[...], preferred_element_type=jnp.float32)
        p = jax.nn.sigmoid(a + b2_ref[...])
        o_ref[...] = jnp.where(t_ref[...] > 0, s, p)

    full = lambda shape: pl.BlockSpec(shape, lambda i: (0, 0))
    return pl.pallas_call(
        body,
        grid=(N // R,),
        in_specs=[
            pl.BlockSpec((R, 1), lambda i: (i, 0)),
            pl.BlockSpec((R, 1), lambda i: (i, 0)),
            full((1, D)),
            full((1, D)),
            full((D, 1)),
            full((1, 1)),
        ],
        out_specs=pl.BlockSpec((R, 1), lambda i: (i, 0)),
        out_shape=jax.ShapeDtypeStruct((N, 1), jnp.float32),
    )(s_col, tm_col, w1row, b1row, w2col, b2s)


def kernel(x, W_emb, b_emb, tok_t, tok_f_real, tok_f_imag,
           Wt1, bt1, Wt2, bt2, Wf1, bf1, Wf2, bf2):
    B, T, C = x.shape
    D = W_emb.shape[0]
    W = _WINDOW
    nmt = int(T * _T_RATIO)
    nmf = int(T * _F_RATIO)

    pe = _pe_table(T, D)
    ex = x @ W_emb.T + b_emb + pe
    exT = jnp.transpose(ex, (0, 2, 1))
    ltr = _win_sum_ref(exT, W)
    ltr2 = _win_sum_ref(exT ** 2, W)
    ltrd = (ltr2 - ltr ** 2)[..., :T]
    ltrm = ltr[..., :T]
    score = ltrd.sum(axis=1) / (ltrm.sum(axis=1) + 1e-6)

    cx = jnp.fft.rfft(exT, axis=-1)
    mag = jnp.sqrt(cx.real ** 2 + cx.imag ** 2)
    day_mag = mag.mean(axis=1)
    Fn = cx.shape[-1]

    kpad = 64
    idx_t_pad, sel_t = _topk_call(score, nmt, kpad)
    idx_t = idx_t_pad[:, :nmt]

    npad = (-Fn) % 128
    dpad = jnp.concatenate(
        [day_mag, jnp.full((B, npad), -jnp.inf, jnp.float32)], axis=1) \
        if npad else day_mag
    idx_f_pad, sel_f = _topk_call(dpad, nmf, kpad)
    idx_f = idx_f_pad[:, :nmf]
    mask_f = sel_f[:, :Fn] > 0

    x2 = x.reshape(B * T, C)
    pe2 = jnp.tile(pe, (B, 1))
    mask2 = sel_t.reshape(B * T, 1)
    temporal_out = _temporal_call(
        x2, pe2, mask2, W_emb, b_emb.reshape(1, D), tok_t.reshape(1, D),
        Wt1, bt1.reshape(1, D), Wt2, bt2.reshape(1, D)).reshape(B, T, D)

    fi = jnp.arange(Fn, dtype=jnp.int32)[:, None]
    ti = jnp.arange(T, dtype=jnp.int32)[None, :]
    angm = ((fi * ti) % T).astype(jnp.float32) * (2.0 * math.pi / T)
    wgt = jnp.where((fi == 0) | (fi == T // 2), 1.0, 2.0).astype(jnp.float32)
    Adft = wgt * jnp.cos(angm) / T
    Bdft = -wgt * jnp.sin(angm) / T

    mxc = _freq_call(
        jnp.real(cx), jnp.imag(cx),
        mask_f.astype(jnp.float32).reshape(B, 1, Fn),
        tok_f_real.reshape(D, 1), tok_f_imag.reshape(D, 1),
        Adft, Bdft, W_emb)

    tm = jnp.fft.irfft(mask_f.astype(jnp.float32), n=T, axis=-1) != 0

    mxc_flat = mxc.reshape(B * T * C, 1)
    tm_flat = jnp.broadcast_to(tm[:, :, None], (B, T, C)) \
        .reshape(B * T * C, 1).astype(jnp.float32)

    def _need_proj(ops):
        return _projf_call(*ops)

    def _skip_proj(ops):
        return ops[0]

    freq_flat = jax.lax.cond(
        jnp.any(jnp.logical_not(tm)), _need_proj, _skip_proj,
        (mxc_flat, tm_flat, Wf1.reshape(1, D), bf1.reshape(1, D),
         Wf2.reshape(D, 1), bf2.reshape(1, 1)))
    freq_out = freq_flat.reshape(B, T, C)

    return temporal_out, idx_t, freq_out, idx_f
